# speedup vs baseline: 1.0032x; 1.0032x over previous
import functools

import jax
import jax.numpy as jnp
from jax.experimental import pallas as pl
from jax.experimental.pallas import tpu as pltpu

N = 10000
E = 320000
SD = 128
NG = 64
NCLS = 41

ROW_BLK = 1000
N_BLKS = N // ROW_BLK


def _layer_body(h_ref, agg_ref, w1_ref, b1_ref, w2_ref, b2_ref, mul_ref, add_ref, out_ref):
    z = h_ref[...] + agg_ref[...]
    z1 = jax.nn.relu(jnp.dot(z, w1_ref[...], preferred_element_type=jnp.float32) + b1_ref[...])
    h2 = jnp.dot(z1, w2_ref[...], preferred_element_type=jnp.float32) + b2_ref[...]
    out_ref[...] = jax.nn.relu(h2) * mul_ref[...] + add_ref[...]


def _gin_dense(h, agg, W1, b1, W2, b2, mul, add):
    full = pl.BlockSpec((SD, SD), lambda i: (0, 0))
    vec = pl.BlockSpec((1, SD), lambda i: (0, 0))
    return pl.pallas_call(
        _layer_body,
        grid=(N_BLKS,),
        in_specs=[
            pl.BlockSpec((ROW_BLK, SD), lambda i: (i, 0)),
            pl.BlockSpec((ROW_BLK, SD), lambda i: (i, 0)),
            full, vec, full, vec, vec, vec,
        ],
        out_specs=pl.BlockSpec((ROW_BLK, SD), lambda i: (i, 0)),
        out_shape=jax.ShapeDtypeStruct((N, SD), jnp.float32),
    )(h, agg, W1, b1.reshape(1, SD), W2, b2.reshape(1, SD),
      mul.reshape(1, SD), add.reshape(1, SD))


def _pool_body(h_ref, batch_ref, wfc1_ref, bfc1_ref, wfc2_ref, bfc2_ref,
               out_ref, acc_ref):
    i = pl.program_id(0)

    @pl.when(i == 0)
    def _():
        acc_ref[...] = jnp.zeros_like(acc_ref)

    b = batch_ref[0, 0, :]
    gids = jax.lax.broadcasted_iota(jnp.int32, (NG, ROW_BLK), 0)
    mask = (gids == b[None, :]).astype(jnp.float32)
    acc_ref[...] += jnp.dot(mask, h_ref[...], preferred_element_type=jnp.float32)

    @pl.when(i == N_BLKS - 1)
    def _():
        p = acc_ref[...]
        hfc = jax.nn.relu(jnp.dot(p, wfc1_ref[...], preferred_element_type=jnp.float32)
                          + bfc1_ref[...])
        out_ref[...] = (jnp.dot(hfc, wfc2_ref[...], preferred_element_type=jnp.float32)
                        + bfc2_ref[...])


def _pool_mlp(h, batch, Wfc1, bfc1, Wfc2, bfc2):
    batch3 = batch.reshape(N_BLKS, 1, ROW_BLK)
    return pl.pallas_call(
        _pool_body,
        grid=(N_BLKS,),
        in_specs=[
            pl.BlockSpec((ROW_BLK, SD), lambda i: (i, 0)),
            pl.BlockSpec((1, 1, ROW_BLK), lambda i: (i, 0, 0)),
            pl.BlockSpec((SD, SD), lambda i: (0, 0)),
            pl.BlockSpec((1, SD), lambda i: (0, 0)),
            pl.BlockSpec((SD, NCLS), lambda i: (0, 0)),
            pl.BlockSpec((1, NCLS), lambda i: (0, 0)),
        ],
        out_specs=pl.BlockSpec((NG, NCLS), lambda i: (0, 0)),
        out_shape=jax.ShapeDtypeStruct((NG, NCLS), jnp.float32),
        scratch_shapes=[pltpu.VMEM((NG, SD), jnp.float32)],
    )(h, batch3, Wfc1, bfc1.reshape(1, SD), Wfc2, bfc2.reshape(1, NCLS))


def kernel(x, edge_index, batch, emb, Win1, bin1, Win2, bin2, g_in, be_in,
           Wh1, bh1, Wh2, bh2, gh, bh, Wo1, bo1, Wo2, bo2,
           Wfc1, bfc1, Wfc2, bfc2):
    src = edge_index[0]
    dst = edge_index[1]
    bnscale = 1.0 / jnp.sqrt(jnp.float32(1.0 + 1e-5))

    h = emb[jnp.squeeze(x, axis=-1)]

    layers = [
        (Win1, bin1, Win2, bin2, g_in * bnscale, be_in),
        (Wh1[0], bh1[0], Wh2[0], bh2[0], gh[0] * bnscale, bh[0]),
        (Wh1[1], bh1[1], Wh2[1], bh2[1], gh[1] * bnscale, bh[1]),
        (Wo1, bo1, Wo2, bo2, jnp.ones((SD,), jnp.float32), jnp.zeros((SD,), jnp.float32)),
    ]
    for (W1, b1, W2, b2, mul, add) in layers:
        agg = jax.ops.segment_sum(h[src], dst, num_segments=N)
        h = _gin_dense(h, agg, W1, b1, W2, b2, mul, add)

    return _pool_mlp(h, batch, Wfc1, bfc1, Wfc2, bfc2)


# speedup vs baseline: 3.8751x; 3.8626x over previous
import functools

import jax
import jax.numpy as jnp
from jax import lax
from jax.experimental import pallas as pl
from jax.experimental.pallas import tpu as pltpu
from jax.experimental.pallas import tpu_sc as plsc

N = 10000
E = 320000
SD = 128
NG = 64
NCLS = 41

NC = 2
NS = 16
NW = NC * NS

NP = 10240
XPW = NP // NW
CPW = 79
EP = NW * CPW * 128
RPZ = NP // NS

ROW_BLK = 1024
N_BLKS = NP // ROW_BLK

_MESH = plsc.VectorSubcoreMesh(core_axis_name="c", subcore_axis_name="s")


@functools.partial(
    pl.kernel,
    out_type=jax.ShapeDtypeStruct((NP, SD), jnp.float32),
    mesh=_MESH,
    scratch_types=[
        pltpu.VMEM((XPW // 64, 64), jnp.int32),
        pltpu.VMEM((XPW, SD), jnp.float32),
        pltpu.SemaphoreType.DMA,
    ],
)
def _emb_gather(x_hbm, emb_hbm, out_hbm, idx_v, rows_v, sem):
    cid = lax.axis_index("c")
    sid = lax.axis_index("s")
    wid = sid * NC + cid
    pltpu.sync_copy(x_hbm.at[wid], idx_v)
    for j in range(XPW // 64):
        pltpu.async_copy(emb_hbm.at[idx_v.at[j]], rows_v.at[pl.ds(j * 64, 64)],
                         sem).wait()
    pltpu.sync_copy(rows_v, out_hbm.at[pl.ds(wid * XPW, XPW)])


@functools.partial(
    pl.kernel,
    out_type=jax.ShapeDtypeStruct((NC, NP, SD), jnp.float32),
    mesh=_MESH,
    scratch_types=[
        pltpu.VMEM((CPW, 128), jnp.int32),
        pltpu.VMEM((CPW, 128), jnp.int32),
        pltpu.VMEM((128, SD), jnp.float32),
        pltpu.VMEM_SHARED((NP, SD), jnp.float32),
        pltpu.SemaphoreType.DMA,
    ],
)
def _sc_agg(h_hbm, src_hbm, dst_hbm, zeros_hbm, out_hbm,
            srcv, dstv, rows, acc, sem):
    cid = lax.axis_index("c")
    sid = lax.axis_index("s")
    wid = sid * NC + cid
    pltpu.sync_copy(zeros_hbm.at[pl.ds(sid * RPZ, RPZ)],
                    acc.at[pl.ds(sid * RPZ, RPZ)])
    pltpu.sync_copy(src_hbm.at[wid], srcv)
    pltpu.sync_copy(dst_hbm.at[wid], dstv)
    plsc.subcore_barrier()

    def body(c, carry):
        pltpu.async_copy(h_hbm.at[srcv.at[c]], rows, sem).wait()
        pltpu.sync_copy(rows, acc.at[dstv.at[c]], add=True)
        return carry

    lax.fori_loop(0, CPW, body, 0)
    plsc.subcore_barrier()
    pltpu.sync_copy(acc.at[pl.ds(sid * RPZ, RPZ)],
                    out_hbm.at[cid].at[pl.ds(sid * RPZ, RPZ)])


def _layer_body(h_ref, p0_ref, p1_ref, w1_ref, b1_ref, w2_ref, b2_ref,
                mul_ref, add_ref, out_ref):
    z = h_ref[...] + p0_ref[...] + p1_ref[...]
    z1 = jax.nn.relu(jnp.dot(z, w1_ref[...], preferred_element_type=jnp.float32)
                     + b1_ref[...])
    h2 = jnp.dot(z1, w2_ref[...], preferred_element_type=jnp.float32) + b2_ref[...]
    out_ref[...] = jax.nn.relu(h2) * mul_ref[...] + add_ref[...]


def _gin_dense(h, p0, p1, W1, b1, W2, b2, mul, add):
    full = pl.BlockSpec((SD, SD), lambda i: (0, 0))
    vec = pl.BlockSpec((1, SD), lambda i: (0, 0))
    row = pl.BlockSpec((ROW_BLK, SD), lambda i: (i, 0))
    return pl.pallas_call(
        _layer_body,
        grid=(N_BLKS,),
        in_specs=[row, row, row, full, vec, full, vec, vec, vec],
        out_specs=row,
        out_shape=jax.ShapeDtypeStruct((NP, SD), jnp.float32),
    )(h, p0, p1, W1, b1.reshape(1, SD), W2, b2.reshape(1, SD),
      mul.reshape(1, SD), add.reshape(1, SD))


def _pool_body(h_ref, batch_ref, wfc1_ref, bfc1_ref, wfc2_ref, bfc2_ref,
               out_ref, acc_ref):
    i = pl.program_id(0)

    @pl.when(i == 0)
    def _():
        acc_ref[...] = jnp.zeros_like(acc_ref)

    b = batch_ref[0, 0, :]
    gids = jax.lax.broadcasted_iota(jnp.int32, (NG, ROW_BLK), 0)
    mask = (gids == b[None, :]).astype(jnp.float32)
    acc_ref[...] += jnp.dot(mask, h_ref[...], preferred_element_type=jnp.float32)

    @pl.when(i == N_BLKS - 1)
    def _():
        p = acc_ref[...]
        hfc = jax.nn.relu(jnp.dot(p, wfc1_ref[...],
                                  preferred_element_type=jnp.float32)
                          + bfc1_ref[...])
        out_ref[...] = (jnp.dot(hfc, wfc2_ref[...],
                                preferred_element_type=jnp.float32)
                        + bfc2_ref[...])


def _pool_mlp(h, batch3, Wfc1, bfc1, Wfc2, bfc2):
    return pl.pallas_call(
        _pool_body,
        grid=(N_BLKS,),
        in_specs=[
            pl.BlockSpec((ROW_BLK, SD), lambda i: (i, 0)),
            pl.BlockSpec((1, 1, ROW_BLK), lambda i: (i, 0, 0)),
            pl.BlockSpec((SD, SD), lambda i: (0, 0)),
            pl.BlockSpec((1, SD), lambda i: (0, 0)),
            pl.BlockSpec((SD, NCLS), lambda i: (0, 0)),
            pl.BlockSpec((1, NCLS), lambda i: (0, 0)),
        ],
        out_specs=pl.BlockSpec((NG, NCLS), lambda i: (0, 0)),
        out_shape=jax.ShapeDtypeStruct((NG, NCLS), jnp.float32),
        scratch_shapes=[pltpu.VMEM((NG, SD), jnp.float32)],
    )(h, batch3, Wfc1, bfc1.reshape(1, SD), Wfc2, bfc2.reshape(1, NCLS))


def kernel(x, edge_index, batch, emb, Win1, bin1, Win2, bin2, g_in, be_in,
           Wh1, bh1, Wh2, bh2, gh, bh, Wo1, bo1, Wo2, bo2,
           Wfc1, bfc1, Wfc2, bfc2):
    src = edge_index[0]
    dst = edge_index[1]
    bnscale = 1.0 / jnp.sqrt(jnp.float32(1.0 + 1e-5))

    x1 = jnp.squeeze(x, axis=-1)
    x3 = jnp.concatenate([x1, jnp.zeros((NP - N,), jnp.int32)]).reshape(
        NW, XPW // 64, 64)
    src3 = jnp.concatenate(
        [src, jnp.zeros((EP - E,), jnp.int32)]).reshape(NW, CPW, 128)
    dst3 = jnp.concatenate(
        [dst, jnp.full((EP - E,), NP - 1, jnp.int32)]).reshape(NW, CPW, 128)
    batch3 = jnp.concatenate(
        [batch, jnp.full((NP - N,), -1, jnp.int32)]).reshape(N_BLKS, 1, ROW_BLK)
    zeros_rows = jnp.zeros((NP, SD), jnp.float32)

    h = _emb_gather(x3, emb)

    layers = [
        (Win1, bin1, Win2, bin2, g_in * bnscale, be_in),
        (Wh1[0], bh1[0], Wh2[0], bh2[0], gh[0] * bnscale, bh[0]),
        (Wh1[1], bh1[1], Wh2[1], bh2[1], gh[1] * bnscale, bh[1]),
        (Wo1, bo1, Wo2, bo2, jnp.ones((SD,), jnp.float32),
         jnp.zeros((SD,), jnp.float32)),
    ]
    for (W1, b1, W2, b2, mul, add) in layers:
        parts = _sc_agg(h, src3, dst3, zeros_rows)
        h = _gin_dense(h, parts[0], parts[1], W1, b1, W2, b2, mul, add)

    return _pool_mlp(h, batch3, Wfc1, bfc1, Wfc2, bfc2)
